# Initial kernel scaffold; baseline (speedup 1.0000x reference)
import jax
import jax.numpy as jnp
from jax.experimental import pallas as pl


def kernel(scores, slices, gn_weight, gn_bias):
    raise NotImplementedError("write your pallas kernel here")


# speedup vs baseline: 1.5096x; 1.5096x over previous
import functools

import jax
import jax.numpy as jnp
from jax import lax
from jax.experimental import pallas as pl
from jax.experimental.pallas import tpu as pltpu
from jax.experimental.pallas import tpu_sc as plsc

B = 64
N = 32768
KSEL = 14
REG_V = 0.5
REG_W = 7
HW = 1024
PAD = 16

NC = 2
NS = 16
NW = NC * NS
RPW = B // NW


NCHK = 256
CLEN = 128
NSEL = 16


def _topk_body(x_ref, idx_ref, val_ref):
    x = x_ref[...]
    rb = x.shape[0]
    s1 = jnp.sum(jnp.sum(x, axis=2), axis=1, keepdims=True)
    mean = (s1 * (1.0 / N))[:, :, None]
    d = x - mean
    ss = jnp.sum(jnp.sum(d * d, axis=2), axis=1, keepdims=True)
    std = jnp.sqrt(ss * (1.0 / (N - 1)))[:, :, None]
    norm = jnp.abs(d) / std
    ci = lax.broadcasted_iota(jnp.int32, (rb, NCHK, CLEN), 1)
    li = lax.broadcasted_iota(jnp.int32, (rb, NCHK, CLEN), 2)
    edge = ((ci == 0) & (li < REG_W)) | ((ci == NCHK - 1) & (li >= CLEN - REG_W))
    norm = jnp.where(edge, norm + REG_V, norm)

    cmax = jnp.max(norm, axis=2)
    cidx = lax.broadcasted_iota(jnp.int32, (rb, NCHK), 1)
    ccs = []
    cm = cmax
    for _ in range(NSEL):
        v = jnp.max(cm, axis=1, keepdims=True)
        c = jnp.min(jnp.where(cm == v, cidx, NCHK), axis=1, keepdims=True)
        ccs.append(c)
        cm = jnp.where(cidx == c, -jnp.inf, cm)
    cc = jnp.concatenate(ccs, axis=1)

    rank = jnp.sum((cc[:, None, :] < cc[:, :, None]).astype(jnp.int32), axis=2)
    rr = lax.broadcasted_iota(jnp.int32, (rb, NSEL, NSEL), 1)
    P = (rank[:, None, :] == rr).astype(jnp.float32)
    sortedcid = jnp.sum(P * cc[:, None, :].astype(jnp.float32),
                        axis=2).astype(jnp.int32)
    cch = lax.broadcasted_iota(jnp.int32, (rb, NSEL, NCHK), 2)
    M = (cc[:, :, None] == cch).astype(jnp.float32)

    cands = []
    for b in range(rb):
        sel = jnp.dot(P[b], M[b], preferred_element_type=jnp.float32,
                      precision=lax.Precision.HIGHEST)
        cands.append(jnp.dot(sel, norm[b],
                             preferred_element_type=jnp.float32,
                             precision=lax.Precision.HIGHEST)[None])
    cand = jnp.concatenate(cands, axis=0)
    colc = sortedcid[:, :, None] * CLEN + lax.broadcasted_iota(
        jnp.int32, (rb, NSEL, CLEN), 2)

    idxs, vals = [], []
    for _ in range(KSEL):
        v2 = jnp.max(jnp.max(cand, axis=2), axis=1)[:, None]
        i2 = jnp.min(jnp.min(jnp.where(cand == v2[:, :, None], colc, N),
                             axis=2), axis=1)[:, None]
        idxs.append(i2)
        vals.append(v2)
        cand = jnp.where(colc == i2[:, :, None], -jnp.inf, cand)
    pad_i = jnp.full((rb, PAD - KSEL), N, jnp.int32)
    pad_v = jnp.zeros((rb, PAD - KSEL), jnp.float32)
    idx_ref[...] = jnp.concatenate(idxs + [pad_i], axis=1)
    val_ref[...] = jnp.concatenate(vals + [pad_v], axis=1)


def _topk_tc(scores):
    rb = 16
    return pl.pallas_call(
        _topk_body,
        grid=(B // rb,),
        in_specs=[pl.BlockSpec((rb, NCHK, CLEN), lambda i: (i, 0, 0))],
        out_specs=[
            pl.BlockSpec((rb, PAD), lambda i: (i, 0)),
            pl.BlockSpec((rb, PAD), lambda i: (i, 0)),
        ],
        out_shape=[
            jax.ShapeDtypeStruct((B, PAD), jnp.int32),
            jax.ShapeDtypeStruct((B, PAD), jnp.float32),
        ],
    )(scores.reshape(B, NCHK, CLEN))


def _rsqrt_newton(x):
    ib = plsc.bitcast(x, jnp.int32)
    ib = 0x5F3759DF - lax.shift_right_logical(ib, 1)
    y = plsc.bitcast(ib, jnp.float32)
    for _ in range(4):
        y = y * (1.5 - 0.5 * x * y * y)
    return y


def _sc_body(idx_hbm, val_hbm, gnw_hbm, gnb_hbm, slices_hbm, out_hbm,
             idx_v, val_v, gnw_v, gnb_v, buf0, buf1, sem0, sem1):
    wid = lax.axis_index("s") * NC + lax.axis_index("c")
    r0 = wid * RPW
    pltpu.sync_copy(gnw_hbm, gnw_v)
    pltpu.sync_copy(gnb_hbm, gnb_v)
    pltpu.sync_copy(idx_hbm.at[pl.ds(r0, RPW)], idx_v)
    pltpu.sync_copy(val_hbm.at[pl.ds(r0, RPW)], val_v)

    lane = lax.broadcasted_iota(jnp.int32, (PAD,), 0)
    mask14 = lane < KSEL
    gnw = gnw_v[...]
    gnb = gnb_v[...]
    bufs = (buf0, buf1)
    sems = (sem0, sem1)

    rows = []
    for j in range(RPW):
        si, sv = plsc.sort_key_val(idx_v[j], val_v[j])
        ig = jnp.where(mask14, si, 0)
        cp = pltpu.async_copy(slices_hbm.at[ig], bufs[j], sems[j])
        rows.append((cp, sv))

    for j in range(RPW):
        cp, sv = rows[j]
        cp.wait()
        buf = bufs[j]
        mn = jnp.full((PAD,), jnp.min(jnp.where(mask14, sv, jnp.inf)))
        mx = jnp.full((PAD,), jnp.max(jnp.where(mask14, sv, -jnp.inf)))
        nss = (sv - mn) / mx * 0.8 + 0.1
        scl = nss * gnw
        bia = nss * gnb

        zero_f = jnp.zeros((PAD,), jnp.float32)
        U = 8
        NCH = HW // 16

        def slice_body(s, carry):
            sm = lane == s
            scl_s = jnp.sum(jnp.where(sm, scl, 0.0))
            bia_s = jnp.sum(jnp.where(sm, bia, 0.0))

            def acc_body(c, acc):
                s1a, s2a, s1b, s2b = acc
                for u in range(U):
                    xv = buf[s, pl.ds((c * U + u) * 16, 16)]
                    if u % 2 == 0:
                        s1a = s1a + xv
                        s2a = s2a + xv * xv
                    else:
                        s1b = s1b + xv
                        s2b = s2b + xv * xv
                return s1a, s2a, s1b, s2b

            s1a, s2a, s1b, s2b = lax.fori_loop(
                0, NCH // U, acc_body, (zero_f, zero_f, zero_f, zero_f))
            t1 = jnp.sum(s1a + s1b)
            t2 = jnp.sum(s2a + s2b)
            mu_s = t1 * (1.0 / HW)
            var_s = t2 * (1.0 / HW) - mu_s * mu_s
            inv = _rsqrt_newton(jnp.full((PAD,), var_s + 1e-5))
            a_v = inv * scl_s
            b_v = bia_s - mu_s * a_v

            def apply_body(c, inner):
                for u in range(U):
                    off = (c * U + u) * 16
                    xv = buf[s, pl.ds(off, 16)]
                    buf[s, pl.ds(off, 16)] = xv * a_v + b_v
                return inner

            lax.fori_loop(0, NCH // U, apply_body, 0)
            return carry

        lax.fori_loop(0, KSEL, slice_body, 0)
        pltpu.sync_copy(buf, out_hbm.at[r0 + j])


def _sc_gather_gn(idx16, val16, gnw16, gnb16, slices2d):
    mesh = plsc.VectorSubcoreMesh(core_axis_name="c", subcore_axis_name="s")
    k = functools.partial(
        pl.kernel,
        mesh=mesh,
        compiler_params=pltpu.CompilerParams(needs_layout_passes=False),
        out_type=jax.ShapeDtypeStruct((B, PAD, HW), jnp.float32),
        scratch_types=[
            pltpu.VMEM((RPW, PAD), jnp.int32),
            pltpu.VMEM((RPW, PAD), jnp.float32),
            pltpu.VMEM((PAD,), jnp.float32),
            pltpu.VMEM((PAD,), jnp.float32),
            pltpu.VMEM((PAD, HW), jnp.float32),
            pltpu.VMEM((PAD, HW), jnp.float32),
            pltpu.SemaphoreType.DMA,
            pltpu.SemaphoreType.DMA,
        ],
    )(_sc_body)
    return k(idx16, val16, gnw16, gnb16, slices2d)


def kernel(scores, slices, gn_weight, gn_bias):
    idx16, val16 = _topk_tc(scores)
    gnw16 = jnp.pad(gn_weight, (0, PAD - KSEL))
    gnb16 = jnp.pad(gn_bias, (0, PAD - KSEL))
    out = _sc_gather_gn(idx16, val16, gnw16, gnb16,
                        slices.reshape(N, HW))
    return out[:, :KSEL].reshape(B, KSEL, 32, 32)


# speedup vs baseline: 2.1764x; 1.4417x over previous
import functools

import jax
import jax.numpy as jnp
from jax import lax
from jax.experimental import pallas as pl
from jax.experimental.pallas import tpu as pltpu
from jax.experimental.pallas import tpu_sc as plsc

B = 64
N = 32768
KSEL = 14
REG_V = 0.5
REG_W = 7
HW = 1024
PAD = 16

NC = 2
NS = 16
NW = NC * NS
RPW = B // NW


NCHK = 256
CLEN = 128
NSEL = 16


def _topk_body(x_ref, idx_ref, val_ref):
    x = x_ref[...]
    rb = x.shape[0]
    s1 = jnp.sum(jnp.sum(x, axis=2), axis=1, keepdims=True)
    mean = (s1 * (1.0 / N))[:, :, None]
    d = x - mean
    ss = jnp.sum(jnp.sum(d * d, axis=2), axis=1, keepdims=True)
    std = jnp.sqrt(ss * (1.0 / (N - 1)))[:, :, None]
    norm = jnp.abs(d) / std
    ci = lax.broadcasted_iota(jnp.int32, (rb, NCHK, CLEN), 1)
    li = lax.broadcasted_iota(jnp.int32, (rb, NCHK, CLEN), 2)
    edge = ((ci == 0) & (li < REG_W)) | ((ci == NCHK - 1) & (li >= CLEN - REG_W))
    norm = jnp.where(edge, norm + REG_V, norm)

    cmax = jnp.max(norm, axis=2)
    cidx = lax.broadcasted_iota(jnp.int32, (rb, NCHK), 1)
    ccs = []
    cm = cmax
    for _ in range(NSEL):
        v = jnp.max(cm, axis=1, keepdims=True)
        c = jnp.min(jnp.where(cm == v, cidx, NCHK), axis=1, keepdims=True)
        ccs.append(c)
        cm = jnp.where(cidx == c, -jnp.inf, cm)
    cc = jnp.concatenate(ccs, axis=1)

    rank = jnp.sum((cc[:, None, :] < cc[:, :, None]).astype(jnp.int32), axis=2)
    rr = lax.broadcasted_iota(jnp.int32, (rb, NSEL, NSEL), 1)
    P = (rank[:, None, :] == rr).astype(jnp.float32)
    sortedcid = jnp.sum(P * cc[:, None, :].astype(jnp.float32),
                        axis=2).astype(jnp.int32)
    cch = lax.broadcasted_iota(jnp.int32, (rb, NSEL, NCHK), 2)
    M = (cc[:, :, None] == cch).astype(jnp.float32)

    cands = []
    for b in range(rb):
        sel = jnp.dot(P[b], M[b], preferred_element_type=jnp.float32,
                      precision=lax.Precision.HIGHEST)
        cands.append(jnp.dot(sel, norm[b],
                             preferred_element_type=jnp.float32,
                             precision=lax.Precision.HIGHEST)[None])
    cand = jnp.concatenate(cands, axis=0)
    colc = sortedcid[:, :, None] * CLEN + lax.broadcasted_iota(
        jnp.int32, (rb, NSEL, CLEN), 2)

    idxs, vals = [], []
    for _ in range(KSEL):
        v2 = jnp.max(jnp.max(cand, axis=2), axis=1)[:, None]
        i2 = jnp.min(jnp.min(jnp.where(cand == v2[:, :, None], colc, N),
                             axis=2), axis=1)[:, None]
        idxs.append(i2)
        vals.append(v2)
        cand = jnp.where(colc == i2[:, :, None], -jnp.inf, cand)
    pad_i = jnp.full((rb, PAD - KSEL), N, jnp.int32)
    pad_v = jnp.zeros((rb, PAD - KSEL), jnp.float32)
    idx_ref[...] = jnp.concatenate(idxs + [pad_i], axis=1)
    val_ref[...] = jnp.concatenate(vals + [pad_v], axis=1)


def _topk_tc(scores):
    rb = 16
    return pl.pallas_call(
        _topk_body,
        grid=(B // rb,),
        in_specs=[pl.BlockSpec((rb, NCHK, CLEN), lambda i: (i, 0, 0))],
        out_specs=[
            pl.BlockSpec((rb, PAD), lambda i: (i, 0)),
            pl.BlockSpec((rb, PAD), lambda i: (i, 0)),
        ],
        out_shape=[
            jax.ShapeDtypeStruct((B, PAD), jnp.int32),
            jax.ShapeDtypeStruct((B, PAD), jnp.float32),
        ],
    )(scores.reshape(B, NCHK, CLEN))


def _rsqrt_newton(x):
    ib = plsc.bitcast(x, jnp.int32)
    ib = 0x5F3759DF - lax.shift_right_logical(ib, 1)
    y = plsc.bitcast(ib, jnp.float32)
    for _ in range(4):
        y = y * (1.5 - 0.5 * x * y * y)
    return y


PGSTRIDE = 256 * HW


def _sc_body(idx_hbm, val_hbm, gnw_hbm, gnb_hbm, slices_hbm, out_hbm,
             idx_v, val_v, gnw_v, gnb_v, ib0, ib1, buf0, buf1, sem0, sem1):
    wid = lax.axis_index("s") * NC + lax.axis_index("c")
    r0 = wid * RPW
    pltpu.sync_copy(gnw_hbm, gnw_v)
    pltpu.sync_copy(gnb_hbm, gnb_v)
    pltpu.sync_copy(idx_hbm.at[pl.ds(r0, RPW)], idx_v)
    pltpu.sync_copy(val_hbm.at[pl.ds(r0, RPW)], val_v)

    lane = lax.broadcasted_iota(jnp.int32, (PAD,), 0)
    mask14 = lane < KSEL
    gnw = gnw_v[...]
    gnb = gnb_v[...]
    bufs = (buf0, buf1)
    ibs = (ib0, ib1)
    sems = (sem0, sem1)
    v16 = lax.shift_right_logical(lane, 3) * PGSTRIDE + (lane & 7) * 128

    rows = []
    for j in range(RPW):
        si, sv = plsc.sort_key_val(idx_v[j], val_v[j])
        ig = jnp.where(mask14, si, 0)
        a_v = lax.shift_right_logical(ig, 7) * 1024 + (ig & 127)
        ib = ibs[j]

        def build_body(s, carry):
            a_s = jnp.full((PAD,), jnp.sum(jnp.where(lane == s, a_v, 0)))
            base = a_s + v16
            for q in range(8):
                for c in range(8):
                    ib[s * 8 + q, pl.ds(c * 16, 16)] = (
                        base + (16 * q + 2 * c) * PGSTRIDE)
            return carry

        lax.fori_loop(0, PAD, build_body, 0)
        cps = [pltpu.async_copy(slices_hbm.at[ib.at[r]], bufs[j].at[r],
                                sems[j]) for r in range(128)]
        rows.append((cps, sv))

    for j in range(RPW):
        cps, sv = rows[j]
        for cp in cps:
            cp.wait()
        buf = bufs[j]
        mn = jnp.full((PAD,), jnp.min(jnp.where(mask14, sv, jnp.inf)))
        mx = jnp.full((PAD,), jnp.max(jnp.where(mask14, sv, -jnp.inf)))
        nss = (sv - mn) / mx * 0.8 + 0.1
        scl = nss * gnw
        bia = nss * gnb

        zero_f = jnp.zeros((PAD,), jnp.float32)
        U = 8
        NCH = HW // 16

        def slice_body(s, carry):
            sm = lane == s
            scl_s = jnp.sum(jnp.where(sm, scl, 0.0))
            bia_s = jnp.sum(jnp.where(sm, bia, 0.0))

            def acc_body(q, acc):
                s1a, s2a, s1b, s2b = acc
                for u in range(U):
                    xv = buf[s * 8 + q, pl.ds(u * 16, 16)]
                    if u % 2 == 0:
                        s1a = s1a + xv
                        s2a = s2a + xv * xv
                    else:
                        s1b = s1b + xv
                        s2b = s2b + xv * xv
                return s1a, s2a, s1b, s2b

            s1a, s2a, s1b, s2b = lax.fori_loop(
                0, 8, acc_body, (zero_f, zero_f, zero_f, zero_f))
            t1 = jnp.sum(s1a + s1b)
            t2 = jnp.sum(s2a + s2b)
            mu_s = t1 * (1.0 / HW)
            var_s = t2 * (1.0 / HW) - mu_s * mu_s
            inv = _rsqrt_newton(jnp.full((PAD,), var_s + 1e-5))
            a_v = inv * scl_s
            b_v = bia_s - mu_s * a_v

            def apply_body(q, inner):
                for u in range(U):
                    xv = buf[s * 8 + q, pl.ds(u * 16, 16)]
                    buf[s * 8 + q, pl.ds(u * 16, 16)] = xv * a_v + b_v
                return inner

            lax.fori_loop(0, 8, apply_body, 0)
            return carry

        lax.fori_loop(0, KSEL, slice_body, 0)
        pltpu.sync_copy(buf, out_hbm.at[r0 + j])


def _sc_gather_gn(idx16, val16, gnw16, gnb16, slices2d):
    mesh = plsc.VectorSubcoreMesh(core_axis_name="c", subcore_axis_name="s")
    k = functools.partial(
        pl.kernel,
        mesh=mesh,
        compiler_params=pltpu.CompilerParams(needs_layout_passes=False),
        out_type=jax.ShapeDtypeStruct((B, 128, 128), jnp.float32),
        scratch_types=[
            pltpu.VMEM((RPW, PAD), jnp.int32),
            pltpu.VMEM((RPW, PAD), jnp.float32),
            pltpu.VMEM((PAD,), jnp.float32),
            pltpu.VMEM((PAD,), jnp.float32),
            pltpu.VMEM((128, 128), jnp.int32),
            pltpu.VMEM((128, 128), jnp.int32),
            pltpu.VMEM((128, 128), jnp.float32),
            pltpu.VMEM((128, 128), jnp.float32),
            pltpu.SemaphoreType.DMA,
            pltpu.SemaphoreType.DMA,
        ],
    )(_sc_body)
    return k(idx16, val16, gnw16, gnb16, slices2d)


def kernel(scores, slices, gn_weight, gn_bias):
    idx16, val16 = _topk_tc(scores)
    gnw16 = jnp.pad(gn_weight, (0, PAD - KSEL))
    gnb16 = jnp.pad(gn_bias, (0, PAD - KSEL))
    slt = slices.reshape(N, HW).T.reshape(128, 8, 256, 128)
    flat = jnp.transpose(slt, (0, 2, 1, 3)).reshape(-1)
    out = _sc_gather_gn(idx16, val16, gnw16, gnb16, flat)
    return (out.reshape(B, PAD, HW)[:, :KSEL]).reshape(B, KSEL, 32, 32)


# speedup vs baseline: 2.3758x; 1.0916x over previous
import functools

import jax
import jax.numpy as jnp
from jax import lax
from jax.experimental import pallas as pl
from jax.experimental.pallas import tpu as pltpu
from jax.experimental.pallas import tpu_sc as plsc

B = 64
N = 32768
KSEL = 14
REG_V = 0.5
REG_W = 7
HW = 1024
PAD = 16

NC = 2
NS = 16
NW = NC * NS
RPW = B // NW


NCHK = 256
CLEN = 128
NSEL = 16


def _topk_body(x_ref, idx_ref, val_ref):
    x = x_ref[...]
    rb = x.shape[0]
    s1 = jnp.sum(jnp.sum(x, axis=2), axis=1, keepdims=True)
    mean = (s1 * (1.0 / N))[:, :, None]
    d = x - mean
    ss = jnp.sum(jnp.sum(d * d, axis=2), axis=1, keepdims=True)
    std = jnp.sqrt(ss * (1.0 / (N - 1)))[:, :, None]
    norm = jnp.abs(d) / std
    ci = lax.broadcasted_iota(jnp.int32, (rb, NCHK, CLEN), 1)
    li = lax.broadcasted_iota(jnp.int32, (rb, NCHK, CLEN), 2)
    edge = ((ci == 0) & (li < REG_W)) | ((ci == NCHK - 1) & (li >= CLEN - REG_W))
    norm = jnp.where(edge, norm + REG_V, norm)

    cmax = jnp.max(norm, axis=2)
    cidx = lax.broadcasted_iota(jnp.int32, (rb, NCHK), 1)
    ccs = []
    cm = cmax
    for _ in range(NSEL):
        v = jnp.max(cm, axis=1, keepdims=True)
        c = jnp.min(jnp.where(cm == v, cidx, NCHK), axis=1, keepdims=True)
        ccs.append(c)
        cm = jnp.where(cidx == c, -jnp.inf, cm)
    cc = jnp.concatenate(ccs, axis=1)

    rank = jnp.sum((cc[:, None, :] < cc[:, :, None]).astype(jnp.int32), axis=2)
    rr = lax.broadcasted_iota(jnp.int32, (rb, NSEL, NSEL), 1)
    P = (rank[:, None, :] == rr).astype(jnp.float32)
    sortedcid = jnp.sum(P * cc[:, None, :].astype(jnp.float32),
                        axis=2).astype(jnp.int32)
    cch = lax.broadcasted_iota(jnp.int32, (rb, NSEL, NCHK), 2)
    M = (cc[:, :, None] == cch).astype(jnp.float32)

    cands = []
    for b in range(rb):
        sel = jnp.dot(P[b], M[b], preferred_element_type=jnp.float32,
                      precision=lax.Precision.HIGHEST)
        cands.append(jnp.dot(sel, norm[b],
                             preferred_element_type=jnp.float32,
                             precision=lax.Precision.HIGHEST)[None])
    cand = jnp.concatenate(cands, axis=0)
    colc = sortedcid[:, :, None] * CLEN + lax.broadcasted_iota(
        jnp.int32, (rb, NSEL, CLEN), 2)

    idxs, vals = [], []
    for _ in range(KSEL):
        v2 = jnp.max(jnp.max(cand, axis=2), axis=1)[:, None]
        i2 = jnp.min(jnp.min(jnp.where(cand == v2[:, :, None], colc, N),
                             axis=2), axis=1)[:, None]
        idxs.append(i2)
        vals.append(v2)
        cand = jnp.where(colc == i2[:, :, None], -jnp.inf, cand)
    pad_i = jnp.full((rb, PAD - KSEL), N, jnp.int32)
    pad_v = jnp.zeros((rb, PAD - KSEL), jnp.float32)
    idx_ref[...] = jnp.concatenate(idxs + [pad_i], axis=1)
    val_ref[...] = jnp.concatenate(vals + [pad_v], axis=1)


def _topk_tc(scores):
    rb = 32
    return pl.pallas_call(
        _topk_body,
        grid=(B // rb,),
        in_specs=[pl.BlockSpec((rb, NCHK, CLEN), lambda i: (i, 0, 0))],
        out_specs=[
            pl.BlockSpec((rb, PAD), lambda i: (i, 0)),
            pl.BlockSpec((rb, PAD), lambda i: (i, 0)),
        ],
        out_shape=[
            jax.ShapeDtypeStruct((B, PAD), jnp.int32),
            jax.ShapeDtypeStruct((B, PAD), jnp.float32),
        ],
    )(scores.reshape(B, NCHK, CLEN))


def _rsqrt_newton(x):
    ib = plsc.bitcast(x, jnp.int32)
    ib = 0x5F3759DF - lax.shift_right_logical(ib, 1)
    y = plsc.bitcast(ib, jnp.float32)
    for _ in range(4):
        y = y * (1.5 - 0.5 * x * y * y)
    return y


PGSTRIDE = 256 * HW


def _sc_body(idx_hbm, val_hbm, gnw_hbm, gnb_hbm, slices_hbm, out_hbm,
             idx_v, val_v, gnw_v, gnb_v, ib0, ib1, buf0, buf1, sem0, sem1):
    wid = lax.axis_index("s") * NC + lax.axis_index("c")
    r0 = wid * RPW
    pltpu.sync_copy(gnw_hbm, gnw_v)
    pltpu.sync_copy(gnb_hbm, gnb_v)
    pltpu.sync_copy(idx_hbm.at[pl.ds(r0, RPW)], idx_v)
    pltpu.sync_copy(val_hbm.at[pl.ds(r0, RPW)], val_v)

    lane = lax.broadcasted_iota(jnp.int32, (PAD,), 0)
    mask14 = lane < KSEL
    gnw = gnw_v[...]
    gnb = gnb_v[...]
    bufs = (buf0, buf1)
    ibs = (ib0, ib1)
    sems = (sem0, sem1)
    v16 = lax.shift_right_logical(lane, 3) * PGSTRIDE + (lane & 7) * 128

    rows = []
    for j in range(RPW):
        si, sv = plsc.sort_key_val(idx_v[j], val_v[j])
        ig = jnp.where(mask14, si, 0)
        a_v = lax.shift_right_logical(ig, 7) * 1024 + (ig & 127)
        ib = ibs[j]

        def build_body(s, carry):
            a_s = jnp.full((PAD,), jnp.sum(jnp.where(lane == s, a_v, 0)))
            base = a_s + v16
            for q in range(8):
                for c in range(8):
                    ib[s * 8 + q, pl.ds(c * 16, 16)] = (
                        base + (16 * q + 2 * c) * PGSTRIDE)
            return carry

        lax.fori_loop(0, PAD, build_body, 0)
        cps = [pltpu.async_copy(slices_hbm.at[ib.at[r]], bufs[j].at[r],
                                sems[j]) for r in range(128)]
        rows.append((cps, sv))

    for j in range(RPW):
        cps, sv = rows[j]
        for cp in cps:
            cp.wait()
        buf = bufs[j]
        mn = jnp.full((PAD,), jnp.min(jnp.where(mask14, sv, jnp.inf)))
        mx = jnp.full((PAD,), jnp.max(jnp.where(mask14, sv, -jnp.inf)))
        nss = (sv - mn) / mx * 0.8 + 0.1
        scl = nss * gnw
        bia = nss * gnb

        zero_f = jnp.zeros((PAD,), jnp.float32)
        U = 8
        NCH = HW // 16

        def slice_body(s, carry):
            sm = lane == s
            scl_s = jnp.sum(jnp.where(sm, scl, 0.0))
            bia_s = jnp.sum(jnp.where(sm, bia, 0.0))

            def acc_body(q, acc):
                s1a, s2a, s1b, s2b = acc
                for u in range(U):
                    xv = buf[s * 8 + q, pl.ds(u * 16, 16)]
                    if u % 2 == 0:
                        s1a = s1a + xv
                        s2a = s2a + xv * xv
                    else:
                        s1b = s1b + xv
                        s2b = s2b + xv * xv
                return s1a, s2a, s1b, s2b

            s1a, s2a, s1b, s2b = lax.fori_loop(
                0, 8, acc_body, (zero_f, zero_f, zero_f, zero_f))
            t1 = jnp.sum(s1a + s1b)
            t2 = jnp.sum(s2a + s2b)
            mu_s = t1 * (1.0 / HW)
            var_s = t2 * (1.0 / HW) - mu_s * mu_s
            inv = _rsqrt_newton(jnp.full((PAD,), var_s + 1e-5))
            a_v = inv * scl_s
            b_v = bia_s - mu_s * a_v

            def apply_body(q, inner):
                for u in range(U):
                    xv = buf[s * 8 + q, pl.ds(u * 16, 16)]
                    buf[s * 8 + q, pl.ds(u * 16, 16)] = xv * a_v + b_v
                return inner

            lax.fori_loop(0, 8, apply_body, 0)
            return carry

        lax.fori_loop(0, KSEL, slice_body, 0)
        pltpu.sync_copy(buf, out_hbm.at[r0 + j])


def _sc_gather_gn(idx16, val16, gnw16, gnb16, slices2d):
    mesh = plsc.VectorSubcoreMesh(core_axis_name="c", subcore_axis_name="s")
    k = functools.partial(
        pl.kernel,
        mesh=mesh,
        compiler_params=pltpu.CompilerParams(needs_layout_passes=False),
        out_type=jax.ShapeDtypeStruct((B, 128, 128), jnp.float32),
        scratch_types=[
            pltpu.VMEM((RPW, PAD), jnp.int32),
            pltpu.VMEM((RPW, PAD), jnp.float32),
            pltpu.VMEM((PAD,), jnp.float32),
            pltpu.VMEM((PAD,), jnp.float32),
            pltpu.VMEM((128, 128), jnp.int32),
            pltpu.VMEM((128, 128), jnp.int32),
            pltpu.VMEM((128, 128), jnp.float32),
            pltpu.VMEM((128, 128), jnp.float32),
            pltpu.SemaphoreType.DMA,
            pltpu.SemaphoreType.DMA,
        ],
    )(_sc_body)
    return k(idx16, val16, gnw16, gnb16, slices2d)


def kernel(scores, slices, gn_weight, gn_bias):
    idx16, val16 = _topk_tc(scores)
    gnw16 = jnp.pad(gn_weight, (0, PAD - KSEL))
    gnb16 = jnp.pad(gn_bias, (0, PAD - KSEL))
    slt = slices.reshape(N, HW).T.reshape(128, 8, 256, 128)
    flat = jnp.transpose(slt, (0, 2, 1, 3)).reshape(-1)
    out = _sc_gather_gn(idx16, val16, gnw16, gnb16, flat)
    return (out.reshape(B, PAD, HW)[:, :KSEL]).reshape(B, KSEL, 32, 32)
